# baseline (speedup 1.0000x reference)
import math

import jax
import jax.numpy as jnp
from jax import lax
from jax.experimental import pallas as pl
from jax.experimental.pallas import tpu as pltpu
from jax.experimental.pallas import tpu_sc as plsc

N = 10000
E = 320000
IN_DIM = 128
H = 8
D = 16
EPS = 1e-09
SCALE = 1.0 / math.sqrt(D)

NC = 2
NS = 16
NW = NC * NS
EPW = E // NW
CH = 40
NCHUNK = EPW // CH
IB = 8
NCHUNK_PAD = 256
NBLK = NCHUNK_PAD // IB
WB = 40
NWB = N // WB
WB_PER_TILE = -(-NWB // NS)
CW = H * D + D


def _proj_body(x_ref, wq_ref, bq_ref, wk_ref, bk_ref, wv_ref, bv_ref,
               q_ref, kv_ref):
    xb = x_ref[...]
    q_ref[...] = jnp.dot(xb, wq_ref[...],
                         preferred_element_type=jnp.float32) + bq_ref[...]
    kv_ref[:, :H * D] = jnp.dot(xb, wk_ref[...],
                                preferred_element_type=jnp.float32) + bk_ref[...]
    kv_ref[:, H * D:] = jnp.dot(xb, wv_ref[...],
                                preferred_element_type=jnp.float32) + bv_ref[...]


def _node_proj(x, WQ, bQ, WK, bK, WV, bV):
    blk = 1000
    grid = N // blk
    wspec = pl.BlockSpec((IN_DIM, H * D), lambda i: (0, 0))
    bspec = pl.BlockSpec((1, H * D), lambda i: (0, 0))
    return pl.pallas_call(
        _proj_body,
        grid=(grid,),
        in_specs=[pl.BlockSpec((blk, IN_DIM), lambda i: (i, 0)),
                  wspec, bspec, wspec, bspec, wspec, bspec],
        out_specs=[pl.BlockSpec((blk, H * D), lambda i: (i, 0)),
                   pl.BlockSpec((blk, 2 * H * D), lambda i: (i, 0))],
        out_shape=[jax.ShapeDtypeStruct((N, H * D), jnp.float32),
                   jax.ShapeDtypeStruct((N, 2 * H * D), jnp.float32)],
    )(x, WQ, bQ.reshape(1, -1), WK, bK.reshape(1, -1), WV, bV.reshape(1, -1))


def _edge_proj_body(ea_ref, we_ref, be_ref, eh_ref):
    eh_ref[...] = jnp.dot(ea_ref[...], we_ref[...],
                          preferred_element_type=jnp.float32) + be_ref[...]


def _edge_proj(edge_attr, WE, bE):
    blk = 4000
    grid = E // blk
    return pl.pallas_call(
        _edge_proj_body,
        grid=(grid,),
        in_specs=[pl.BlockSpec((blk, IN_DIM), lambda i: (i, 0)),
                  pl.BlockSpec((IN_DIM, H * D), lambda i: (0, 0)),
                  pl.BlockSpec((1, H * D), lambda i: (0, 0))],
        out_specs=pl.BlockSpec((blk, H * D), lambda i: (i, 0)),
        out_shape=jax.ShapeDtypeStruct((E, H * D), jnp.float32),
    )(edge_attr, WE, bE.reshape(1, -1))


def _edge_kernel6(src_hbm, dst_hbm, qh_hbm, kvh_hbm, eh_hbm,
                  oacc_hbm,
                  sidx, didx, kv_v, q_v, e_v, c_v,
                  semg0, semg1, semq,
                  acc_sh):
    cid = lax.axis_index("c")
    sid = lax.axis_index("s")
    wid = sid * NC + cid
    semg = (semg0, semg1)

    zero16 = jnp.zeros((16,), jnp.float32)

    def _zrow(r, carry):
        for cc in range(CW // 16):
            c_v[r, pl.ds(cc * 16, 16)] = zero16
        return carry
    lax.fori_loop(0, WB, _zrow, 0)

    def _initj(j, carry):
        ci = sid + j * NS
        @pl.when(ci < NWB)
        def _init():
            r0 = pl.multiple_of(ci * WB, 8)
            pltpu.sync_copy(c_v, acc_sh.at[pl.ds(r0, WB)])
        return carry
    lax.fori_loop(0, WB_PER_TILE, _initj, 0)
    plsc.subcore_barrier()

    lane = lax.iota(jnp.int32, 16)
    onehot = [(lane == h).astype(jnp.float32) for h in range(H)]
    perms = [lane ^ st for st in (8, 4, 2, 1)]
    bidx = [lane * 0 + h for h in range(H)]

    ebase = wid * EPW

    def _load_idx(c, s):
        e0 = ebase + c * CH
        pltpu.sync_copy(src_hbm.at[pl.ds(e0, CH)], sidx.at[s])
        pltpu.sync_copy(dst_hbm.at[pl.ds(e0, CH)], didx.at[s])

    def _fire_kv(s, b):
        pltpu.async_copy(kvh_hbm.at[sidx.at[s]], kv_v.at[b], semg[b])

    def _fire_q(s):
        pltpu.async_copy(qh_hbm.at[didx.at[s]], q_v, semq)

    def _wait_kv(b):
        pltpu.make_async_copy(kvh_hbm.at[sidx.at[0]], kv_v.at[b], semg[b]).wait()

    def _wait_q():
        pltpu.make_async_copy(qh_hbm.at[didx.at[0]], q_v, semq).wait()

    def _stage(c, b, prefetch):
        if prefetch:
            _load_idx(c + 1, 1 - b)
            _fire_kv(1 - b, 1 - b)
        e0 = ebase + c * CH
        pltpu.sync_copy(eh_hbm.at[pl.ds(e0, CH)], e_v)
        _wait_kv(b)
        _wait_q()
        kvb = kv_v.at[b]

        def _dots(e, ecarry):
            p = [kvb[e, pl.ds(h * 16, 16)] * q_v[e, pl.ds(h * 16, 16)]
                 * e_v[e, pl.ds(h * 16, 16)] for h in range(H)]
            for pm in perms:
                p = [ph + jnp.take(ph, pm) for ph in p]
            srow = p[0] * onehot[0]
            for h in range(1, H):
                srow = srow + p[h] * onehot[h]
            svec_all = jnp.exp(srow * SCALE)
            c_v[e, pl.ds(H * D, 16)] = svec_all
            for h in range(H):
                sv = jnp.take(svec_all, bidx[h])
                c_v[e, pl.ds(h * 16, 16)] = (
                    kvb[e, pl.ds(H * D + h * 16, 16)] * sv)
            return ecarry
        lax.fori_loop(0, CH, _dots, 0)

        if prefetch:
            _fire_q(1 - b)
        pltpu.sync_copy(c_v, acc_sh.at[didx.at[b]], add=True)

    _load_idx(0, 0)
    _fire_kv(0, 0)
    _fire_q(0)

    def _pair(i, carry):
        c0 = i * 2
        _stage(c0, 0, True)
        _stage(c0 + 1, 1, True)
        return carry
    lax.fori_loop(0, NCHUNK // 2 - 1, _pair, 0)
    _stage(NCHUNK - 2, 0, True)
    _stage(NCHUNK - 1, 1, False)

    plsc.subcore_barrier()

    def _wbj(j, carry):
        ci = sid + j * NS
        @pl.when(ci < NWB)
        def _wb():
            r0 = pl.multiple_of(ci * WB, 8)
            ro = pl.multiple_of(cid * N + r0, 8)
            pltpu.sync_copy(acc_sh.at[pl.ds(r0, WB)], c_v)
            pltpu.sync_copy(c_v, oacc_hbm.at[pl.ds(ro, WB)])
        return carry
    lax.fori_loop(0, WB_PER_TILE, _wbj, 0)


def _edge_stage6(src, dst, Qh, KVh, Eh):
    mesh = plsc.VectorSubcoreMesh(core_axis_name="c", subcore_axis_name="s")
    f = pl.kernel(
        _edge_kernel6,
        out_type=[jax.ShapeDtypeStruct((NC * N, CW), jnp.float32)],
        mesh=mesh,
        compiler_params=pltpu.CompilerParams(needs_layout_passes=False,
                                             use_tc_tiling_on_sc=False),
        scratch_types=[
            pltpu.VMEM((2, CH), jnp.int32),
            pltpu.VMEM((2, CH), jnp.int32),
            pltpu.VMEM((2, CH, 2 * H * D), jnp.float32),
            pltpu.VMEM((CH, H * D), jnp.float32),
            pltpu.VMEM((CH, H * D), jnp.float32),
            pltpu.VMEM((CH, CW), jnp.float32),
            pltpu.SemaphoreType.DMA,
            pltpu.SemaphoreType.DMA,
            pltpu.SemaphoreType.DMA,
            pltpu.VMEM_SHARED((N, CW), jnp.float32),
        ],
    )
    return f(src, dst, Qh, KVh, Eh)


def _final_body(acc_ref, out_ref):
    acc = acc_ref[0] + acc_ref[1]
    for h in range(H):
        denom = acc[:, H * D + h:H * D + h + 1] + EPS
        out_ref[:, h * D:(h + 1) * D] = acc[:, h * D:(h + 1) * D] / denom


def _finalize(oacc):
    blk = 1000
    grid = N // blk
    acc2 = oacc.reshape(NC, N, CW)
    return pl.pallas_call(
        _final_body,
        grid=(grid,),
        in_specs=[pl.BlockSpec((NC, blk, CW), lambda i: (0, i, 0))],
        out_specs=pl.BlockSpec((blk, H * D), lambda i: (i, 0)),
        out_shape=jax.ShapeDtypeStruct((N, H * D), jnp.float32),
    )(acc2)


def kernel(x, edge_attr, edge_index, WQ, bQ, WK, bK, WV, bV, WE, bE):
    Qh, KVh = _node_proj(x, WQ, bQ, WK, bK, WV, bV)
    Eh = _edge_proj(edge_attr, WE, bE)
    src = edge_index[0]
    dst = edge_index[1]
    oacc, = _edge_stage6(src, dst, Qh, KVh, Eh)
    out = _finalize(oacc)
    return out.reshape(N, H, D)


# speedup vs baseline: 1.1832x; 1.1832x over previous
import math

import jax
import jax.numpy as jnp
from jax import lax
from jax.experimental import pallas as pl
from jax.experimental.pallas import tpu as pltpu
from jax.experimental.pallas import tpu_sc as plsc

N = 10000
E = 320000
IN_DIM = 128
H = 8
D = 16
EPS = 1e-09
SCALE = 1.0 / math.sqrt(D)

NC = 2
NS = 16
NW = NC * NS
EPW = E // NW
CH = 40
NCHUNK = EPW // CH
WB = 80
NWB = N // WB
WB_PER_TILE = -(-NWB // NS)


def _proj_body(x_ref, wq_ref, bq_ref, wk_ref, bk_ref, wv_ref, bv_ref,
               q_ref, k_ref, v_ref):
    xb = x_ref[...]
    q_ref[...] = jnp.dot(xb, wq_ref[...],
                         preferred_element_type=jnp.float32) + bq_ref[...]
    k_ref[...] = jnp.dot(xb, wk_ref[...],
                         preferred_element_type=jnp.float32) + bk_ref[...]
    v_ref[...] = jnp.dot(xb, wv_ref[...],
                         preferred_element_type=jnp.float32) + bv_ref[...]


def _node_proj(x, WQ, bQ, WK, bK, WV, bV):
    blk = 1000
    grid = N // blk
    wspec = pl.BlockSpec((IN_DIM, H * D), lambda i: (0, 0))
    bspec = pl.BlockSpec((1, H * D), lambda i: (0, 0))
    ospec = pl.BlockSpec((blk, H * D), lambda i: (i, 0))
    return pl.pallas_call(
        _proj_body,
        grid=(grid,),
        in_specs=[pl.BlockSpec((blk, IN_DIM), lambda i: (i, 0)),
                  wspec, bspec, wspec, bspec, wspec, bspec],
        out_specs=[ospec, ospec, ospec],
        out_shape=[jax.ShapeDtypeStruct((N, H * D), jnp.float32)] * 3,
    )(x, WQ, bQ.reshape(1, -1), WK, bK.reshape(1, -1), WV, bV.reshape(1, -1))


def _edge_proj_body(ea_ref, we_ref, be_ref, eh_ref):
    eh_ref[...] = jnp.dot(ea_ref[...], we_ref[...],
                          preferred_element_type=jnp.float32) + be_ref[...]


def _edge_proj(edge_attr, WE, bE):
    blk = 4000
    grid = E // blk
    return pl.pallas_call(
        _edge_proj_body,
        grid=(grid,),
        in_specs=[pl.BlockSpec((blk, IN_DIM), lambda i: (i, 0)),
                  pl.BlockSpec((IN_DIM, H * D), lambda i: (0, 0)),
                  pl.BlockSpec((1, H * D), lambda i: (0, 0))],
        out_specs=pl.BlockSpec((blk, H * D), lambda i: (i, 0)),
        out_shape=jax.ShapeDtypeStruct((E, H * D), jnp.float32),
    )(edge_attr, WE, bE.reshape(1, -1))


def _edge_kernel(src_hbm, dst_hbm, qh_hbm, kh_hbm, vh_hbm, eh_hbm,
                 owv_hbm, owz_hbm,
                 src_v, dst_v, k_v, q_v, v_v, e_v, s_v, wb_v, wbz_v,
                 sem, wv_sh, wz_sh):
    cid = lax.axis_index("c")
    sid = lax.axis_index("s")
    wid = sid * NC + cid

    zero16 = jnp.zeros((16,), jnp.float32)

    def _zrow(r, carry):
        for cc in range(8):
            wb_v[r, pl.ds(cc * 16, 16)] = zero16
        wbz_v[r, :] = zero16
        return carry
    lax.fori_loop(0, WB, _zrow, 0)

    for j in range(WB_PER_TILE):
        ci = sid + j * NS
        @pl.when(ci < NWB)
        def _init():
            r0 = ci * WB
            pltpu.sync_copy(wb_v, wv_sh.at[pl.ds(r0, WB)])
            pltpu.sync_copy(wbz_v, wz_sh.at[pl.ds(r0, WB)])
    plsc.subcore_barrier()

    lane = lax.iota(jnp.int32, 16)
    onehot = [(lane == h).astype(jnp.float32) for h in range(H)]
    perms = [lane ^ st for st in (8, 4, 2, 1)]
    bidx = [lane * 0 + h for h in range(H)]

    ebase = wid * EPW

    def _chunk(c, carry):
        e0 = ebase + c * CH
        pltpu.sync_copy(src_hbm.at[pl.ds(e0, CH)], src_v)
        pltpu.sync_copy(dst_hbm.at[pl.ds(e0, CH)], dst_v)
        cp_k = pltpu.async_copy(kh_hbm.at[src_v], k_v, sem)
        cp_q = pltpu.async_copy(qh_hbm.at[dst_v], q_v, sem)
        cp_v = pltpu.async_copy(vh_hbm.at[src_v], v_v, sem)
        pltpu.sync_copy(eh_hbm.at[pl.ds(e0, CH)], e_v)
        cp_k.wait()
        cp_q.wait()
        cp_v.wait()

        def _edge(e, ecarry):
            p = [k_v[e, pl.ds(h * 16, 16)] * q_v[e, pl.ds(h * 16, 16)]
                 * e_v[e, pl.ds(h * 16, 16)] for h in range(H)]
            for pm in perms:
                p = [ph + jnp.take(ph, pm) for ph in p]
            srow = p[0] * onehot[0]
            for h in range(1, H):
                srow = srow + p[h] * onehot[h]
            svec_all = jnp.exp(srow * SCALE)
            s_v[e, :] = svec_all
            for h in range(H):
                sv = jnp.take(svec_all, bidx[h])
                v_v[e, pl.ds(h * 16, 16)] = v_v[e, pl.ds(h * 16, 16)] * sv
            return ecarry
        lax.fori_loop(0, CH, _edge, 0)

        pltpu.sync_copy(v_v, wv_sh.at[dst_v], add=True)
        pltpu.sync_copy(s_v, wz_sh.at[dst_v], add=True)
        return carry
    lax.fori_loop(0, NCHUNK, _chunk, 0)

    plsc.subcore_barrier()

    for j in range(WB_PER_TILE):
        ci = sid + j * NS
        @pl.when(ci < NWB)
        def _wb():
            r0 = ci * WB
            pltpu.sync_copy(wv_sh.at[pl.ds(r0, WB)], wb_v)
            pltpu.sync_copy(wb_v, owv_hbm.at[pl.ds(cid * N + r0, WB)])
            pltpu.sync_copy(wz_sh.at[pl.ds(r0, WB)], wbz_v)
            pltpu.sync_copy(wbz_v, owz_hbm.at[pl.ds(cid * N + r0, WB)])


def _edge_stage(src, dst, Qh, Kh, Vh, Eh):
    mesh = plsc.VectorSubcoreMesh(core_axis_name="c", subcore_axis_name="s")
    f = pl.kernel(
        _edge_kernel,
        out_type=[jax.ShapeDtypeStruct((NC * N, H * D), jnp.float32),
                  jax.ShapeDtypeStruct((NC * N, D), jnp.float32)],
        mesh=mesh,
        compiler_params=pltpu.CompilerParams(needs_layout_passes=False,
                                             use_tc_tiling_on_sc=False),
        scratch_types=[
            pltpu.VMEM((CH,), jnp.int32),
            pltpu.VMEM((CH,), jnp.int32),
            pltpu.VMEM((CH, H * D), jnp.float32),
            pltpu.VMEM((CH, H * D), jnp.float32),
            pltpu.VMEM((CH, H * D), jnp.float32),
            pltpu.VMEM((CH, H * D), jnp.float32),
            pltpu.VMEM((CH, D), jnp.float32),
            pltpu.VMEM((WB, H * D), jnp.float32),
            pltpu.VMEM((WB, D), jnp.float32),
            pltpu.SemaphoreType.DMA,
            pltpu.VMEM_SHARED((N, H * D), jnp.float32),
            pltpu.VMEM_SHARED((N, D), jnp.float32),
        ],
    )
    return f(src, dst, Qh, Kh, Vh, Eh)


def _final_body(wv_ref, wz_ref, out_ref):
    wv = wv_ref[0] + wv_ref[1]
    wz = wz_ref[0] + wz_ref[1]
    for h in range(H):
        denom = wz[:, h:h + 1] + EPS
        out_ref[:, h * D:(h + 1) * D] = wv[:, h * D:(h + 1) * D] / denom


def _finalize(owv, owz):
    blk = 1000
    grid = N // blk
    wv2 = owv.reshape(NC, N, H * D)
    wz2 = owz.reshape(NC, N, D)
    return pl.pallas_call(
        _final_body,
        grid=(grid,),
        in_specs=[pl.BlockSpec((NC, blk, H * D), lambda i: (0, i, 0)),
                  pl.BlockSpec((NC, blk, D), lambda i: (0, i, 0))],
        out_specs=pl.BlockSpec((blk, H * D), lambda i: (i, 0)),
        out_shape=jax.ShapeDtypeStruct((N, H * D), jnp.float32),
    )(wv2, wz2)


def kernel(x, edge_attr, edge_index, WQ, bQ, WK, bK, WV, bV, WE, bE):
    Qh, Kh, Vh = _node_proj(x, WQ, bQ, WK, bK, WV, bV)
    Eh = _edge_proj(edge_attr, WE, bE)
    src = edge_index[0]
    dst = edge_index[1]
    owv, owz = _edge_stage(src, dst, Qh, Kh, Vh, Eh)
    out = _finalize(owv, owz)
    return out.reshape(N, H, D)
